# baseline (speedup 1.0000x reference)
import jax
import jax.numpy as jnp
from jax import lax
from jax.experimental import pallas as pl
from jax.experimental.pallas import tpu as pltpu
from jax.experimental.pallas import tpu_sc as plsc

_B, _Q, _C, _D = 4096, 50, 200, 32
_NC, _NS, _L = 2, 16, 16
_NW = _NC * _NS
_RPW = _B // _NW
_CH = 100

_PASS1 = (0, 16, 32, 48, 64, 80, 96)
_PASS2 = (112, 128, 144, 160, 176, 184)


def _splat(i):
    return jnp.full((_L,), i, jnp.int32)


def _rsqrt(n):
    i = plsc.bitcast(n, jnp.int32)
    i = jnp.full((_L,), 0x5F3759DF, jnp.int32) - lax.shift_right_logical(i, 1)
    y = plsc.bitcast(i, jnp.float32)
    for _ in range(3):
        y = y * (1.5 - 0.5 * n * y * y)
    return y


def _body(pidx_hbm, qidx_hbm, cidx_hbm, sym_hbm, pat_hbm, par_hbm, out_hbm,
          pidx_v, qidx_v, cidx_v, prow_v, qbuf_v, cbuf_v, outb_v,
          par_v, red_v, sem):
    wid = lax.axis_index("s") * _NC + lax.axis_index("c")
    base = wid * _RPW

    pltpu.sync_copy(par_hbm, par_v)
    pltpu.sync_copy(pidx_hbm.at[pl.ds(base, _RPW)], pidx_v)
    pltpu.sync_copy(qidx_hbm.at[pl.ds(base, _RPW)], qidx_v)
    pltpu.sync_copy(cidx_hbm.at[pl.ds(base, _RPW)], cidx_v)
    pltpu.async_copy(pat_hbm.at[pidx_v], prow_v, sem).wait()

    par = par_v[...]
    w0v = lax.broadcast(par[0], (_L,))
    w1hv = lax.broadcast(par[1], (_L,))
    biasv = lax.broadcast(par[2], (_L,))
    iota = lax.iota(jnp.int32, _L)
    zf = jnp.zeros((_L,), jnp.float32)

    def issue_row(b, par):
        pltpu.async_copy(sym_hbm.at[qidx_v.at[b]], qbuf_v.at[par], sem)
        pltpu.async_copy(sym_hbm.at[cidx_v.at[b, 0]],
                         cbuf_v.at[par, pl.ds(0, _CH)], sem)
        pltpu.async_copy(sym_hbm.at[cidx_v.at[b, 1]],
                         cbuf_v.at[par, pl.ds(_CH, _CH)], sem)

    issue_row(0, 0)

    def row_step(b, carry):
        par = lax.bitwise_and(b, 1)
        pltpu.make_async_copy(sym_hbm.at[qidx_v.at[b]],
                              qbuf_v.at[par], sem).wait()
        pltpu.make_async_copy(sym_hbm.at[cidx_v.at[b, 0]],
                              cbuf_v.at[par, pl.ds(0, _CH)], sem).wait()
        pltpu.make_async_copy(sym_hbm.at[cidx_v.at[b, 1]],
                              cbuf_v.at[par, pl.ds(_CH, _CH)], sem).wait()

        @pl.when(b < _RPW - 1)
        def _():
            issue_row(b + 1, 1 - par)

        def qsum(j, acc):
            return (acc[0] + qbuf_v[par, j, pl.ds(0, _L)],
                    acc[1] + qbuf_v[par, j, pl.ds(_L, _L)])
        qlo, qhi = lax.fori_loop(0, _Q, qsum, (zf, zf))
        s = qlo * qlo + qhi * qhi
        red_v[0, pl.ds(_L, _L)] = zf
        for step in (8, 4, 2, 1):
            red_v[0, pl.ds(0, _L)] = s
            s = s + plsc.load_gather(red_v, [_splat(0), iota + step])
        nq = lax.broadcast(s[0], (_L,))
        rq = jnp.minimum(_rsqrt(nq), 1.0 / (_Q * 1e-12))
        red_v[0, pl.ds(0, _L)] = qlo * rq
        red_v[0, pl.ds(_L, _L)] = qhi * rq

        parv = lax.broadcast(par, (_L,))
        bsplat = lax.broadcast(b, (_L,))
        zsplat = _splat(0)
        for starts in (_PASS1, _PASS2):
            nch = len(starts)

            def tstep(t, carry):
                m = carry[-1]
                accs = carry[:-1]
                pv = plsc.load_gather(prow_v, [bsplat, m])
                qv = plsc.load_gather(red_v, [zsplat, m])
                out = []
                for k in range(nch):
                    ap, aq, an = accs[3 * k:3 * k + 3]
                    ev = plsc.load_gather(cbuf_v,
                                          [parv, iota + starts[k], m])
                    out += [ap + ev * pv, aq + ev * qv, an + ev * ev]
                return out + [lax.bitwise_and(m + 1, _D - 1)]

            accs = lax.fori_loop(0, _D, tstep, [zf] * (3 * nch) + [iota])
            for k in range(nch):
                ap, aq, an = accs[3 * k:3 * k + 3]
                cf = ap + biasv
                sig = 1.0 / (1.0 + jnp.exp(-cf))
                r = jnp.minimum(_rsqrt(an), 1.0e12)
                res = w0v * sig + w1hv * (aq * r) + w1hv
                outb_v[b, pl.ds(starts[k], _L)] = res
        return carry

    lax.fori_loop(0, _RPW, row_step, 0)
    pltpu.sync_copy(outb_v, out_hbm.at[pl.ds(base, _RPW)])


def _make_call(interpret=False):
    mesh = plsc.VectorSubcoreMesh(core_axis_name="c", subcore_axis_name="s")
    return pl.kernel(
        _body,
        out_type=jax.ShapeDtypeStruct((_B, _C), jnp.float32),
        mesh=mesh,
        scratch_types=[
            pltpu.VMEM((_RPW,), jnp.int32),
            pltpu.VMEM((_RPW, _Q), jnp.int32),
            pltpu.VMEM((_RPW, 2, _CH), jnp.int32),
            pltpu.VMEM((_RPW, _D), jnp.float32),
            pltpu.VMEM((2, _Q, _D), jnp.float32),
            pltpu.VMEM((2, _C, _D), jnp.float32),
            pltpu.VMEM((_RPW, _C), jnp.float32),
            pltpu.VMEM((_L,), jnp.float32),
            pltpu.VMEM((1, 2 * _L), jnp.float32),
            pltpu.SemaphoreType.DMA,
        ],
        compiler_params=pltpu.CompilerParams(needs_layout_passes=False,
                                             use_tc_tiling_on_sc=False,
                                             disable_bounds_checks=True),
        interpret=interpret,
    )


def kernel(patient_idx, gender, age_bin, query_symptoms, candidate_symptoms,
           symptom_emb, patient_emb, demog_W, demog_b, cf_bias,
           fusion_weights):
    w = jax.nn.softmax(fusion_weights.astype(jnp.float32))
    params = (jnp.zeros((_L,), jnp.float32)
              .at[0].set(w[0])
              .at[1].set(w[1] * 0.5)
              .at[2].set(cf_bias.astype(jnp.float32)[0]))
    pidx = patient_idx.astype(jnp.int32)
    qidx = query_symptoms.astype(jnp.int32)
    cidx = candidate_symptoms.astype(jnp.int32).reshape(_B, 2, _CH)
    return _make_call()(pidx, qidx, cidx,
                        symptom_emb.astype(jnp.float32),
                        patient_emb.astype(jnp.float32),
                        params)


# speedup vs baseline: 1.0119x; 1.0119x over previous
import jax
import jax.numpy as jnp
from jax import lax
from jax.experimental import pallas as pl
from jax.experimental.pallas import tpu as pltpu
from jax.experimental.pallas import tpu_sc as plsc

_B, _Q, _C, _D = 4096, 50, 200, 32
_NC, _NS, _L = 2, 16, 16
_NW = _NC * _NS
_RPW = _B // _NW
_CH = 100

_PASS1 = (0, 16, 32, 48, 64, 80, 96)
_PASS2 = (112, 128, 144, 160, 176, 184)


def _splat(i):
    return jnp.full((_L,), i, jnp.int32)


def _rsqrt(n):
    i = plsc.bitcast(n, jnp.int32)
    i = jnp.full((_L,), 0x5F3759DF, jnp.int32) - lax.shift_right_logical(i, 1)
    y = plsc.bitcast(i, jnp.float32)
    for _ in range(3):
        y = y * (1.5 - 0.5 * n * y * y)
    return y


_NBUF = 4


def _body(pidx_hbm, qidx_hbm, cidx_hbm, sym_hbm, pat_hbm, par_hbm, out_hbm,
          pidx_v, qidx_v, cidx_v, prow_v, qbuf_v, cbuf_v, outb_v,
          par_v, red_v, sem0, sem1, sem2, sem3):
    sems = (sem0, sem1, sem2, sem3)
    wid = lax.axis_index("s") * _NC + lax.axis_index("c")
    base = wid * _RPW

    pltpu.sync_copy(par_hbm, par_v)
    pltpu.sync_copy(pidx_hbm.at[pl.ds(base, _RPW)], pidx_v)
    pltpu.sync_copy(qidx_hbm.at[pl.ds(base, _RPW)], qidx_v)
    pltpu.sync_copy(cidx_hbm.at[pl.ds(base, _RPW)], cidx_v)
    pltpu.async_copy(pat_hbm.at[pidx_v], prow_v, sem0).wait()

    par = par_v[...]
    w0v = lax.broadcast(par[0], (_L,))
    w1hv = lax.broadcast(par[1], (_L,))
    biasv = lax.broadcast(par[2], (_L,))
    iota = lax.iota(jnp.int32, _L)
    zf = jnp.zeros((_L,), jnp.float32)

    def issue_row(b, slot):
        pltpu.async_copy(sym_hbm.at[qidx_v.at[b]], qbuf_v.at[slot],
                         sems[slot])
        pltpu.async_copy(sym_hbm.at[cidx_v.at[b, 0]],
                         cbuf_v.at[slot, pl.ds(0, _CH)], sems[slot])
        pltpu.async_copy(sym_hbm.at[cidx_v.at[b, 1]],
                         cbuf_v.at[slot, pl.ds(_CH, _CH)], sems[slot])

    def wait_row(b, slot):
        pltpu.make_async_copy(sym_hbm.at[qidx_v.at[b]],
                              qbuf_v.at[slot], sems[slot]).wait()
        pltpu.make_async_copy(sym_hbm.at[cidx_v.at[b, 0]],
                              cbuf_v.at[slot, pl.ds(0, _CH)],
                              sems[slot]).wait()
        pltpu.make_async_copy(sym_hbm.at[cidx_v.at[b, 1]],
                              cbuf_v.at[slot, pl.ds(_CH, _CH)],
                              sems[slot]).wait()

    def compute_row(b, slot):
        def qsum(j, acc):
            return (acc[0] + qbuf_v[slot, j, pl.ds(0, _L)],
                    acc[1] + qbuf_v[slot, j, pl.ds(_L, _L)])
        qlo, qhi = lax.fori_loop(0, _Q, qsum, (zf, zf))
        s = qlo * qlo + qhi * qhi
        red_v[0, pl.ds(_L, _L)] = zf
        for step in (8, 4, 2, 1):
            red_v[0, pl.ds(0, _L)] = s
            s = s + plsc.load_gather(red_v, [_splat(0), iota + step])
        nq = lax.broadcast(s[0], (_L,))
        rq = jnp.minimum(_rsqrt(nq), 1.0 / (_Q * 1e-12))
        red_v[0, pl.ds(0, _L)] = qlo * rq
        red_v[0, pl.ds(_L, _L)] = qhi * rq

        parv = _splat(slot)
        bsplat = lax.broadcast(b, (_L,))
        zsplat = _splat(0)
        for starts in (_PASS1, _PASS2):
            nch = len(starts)

            def tstep(t, carry):
                m = carry[-1]
                accs = carry[:-1]
                pv = plsc.load_gather(prow_v, [bsplat, m])
                qv = plsc.load_gather(red_v, [zsplat, m])
                out = []
                for k in range(nch):
                    ap, aq, an = accs[3 * k:3 * k + 3]
                    ev = plsc.load_gather(cbuf_v,
                                          [parv, iota + starts[k], m])
                    out += [ap + ev * pv, aq + ev * qv, an + ev * ev]
                return out + [lax.bitwise_and(m + 1, _D - 1)]

            accs = lax.fori_loop(0, _D, tstep, [zf] * (3 * nch) + [iota])
            for k in range(nch):
                ap, aq, an = accs[3 * k:3 * k + 3]
                cf = ap + biasv
                sig = 1.0 / (1.0 + jnp.exp(-cf))
                r = jnp.minimum(_rsqrt(an), 1.0e12)
                res = w0v * sig + w1hv * (aq * r) + w1hv
                outb_v[b, pl.ds(starts[k], _L)] = res

    for s in range(_NBUF):
        issue_row(s, s)

    def group_step(g, carry):
        for s in range(_NBUF):
            b = g * _NBUF + s
            wait_row(b, s)
            compute_row(b, s)

            @pl.when(g < _RPW // _NBUF - 1)
            def _(b=b, s=s):
                issue_row(b + _NBUF, s)
        return carry

    lax.fori_loop(0, _RPW // _NBUF, group_step, 0)
    pltpu.sync_copy(outb_v, out_hbm.at[pl.ds(base, _RPW)])


def _make_call(interpret=False):
    mesh = plsc.VectorSubcoreMesh(core_axis_name="c", subcore_axis_name="s")
    return pl.kernel(
        _body,
        out_type=jax.ShapeDtypeStruct((_B, _C), jnp.float32),
        mesh=mesh,
        scratch_types=[
            pltpu.VMEM((_RPW,), jnp.int32),
            pltpu.VMEM((_RPW, _Q), jnp.int32),
            pltpu.VMEM((_RPW, 2, _CH), jnp.int32),
            pltpu.VMEM((_RPW, _D), jnp.float32),
            pltpu.VMEM((_NBUF, _Q, _D), jnp.float32),
            pltpu.VMEM((_NBUF, _C, _D), jnp.float32),
            pltpu.VMEM((_RPW, _C), jnp.float32),
            pltpu.VMEM((_L,), jnp.float32),
            pltpu.VMEM((1, 2 * _L), jnp.float32),
            pltpu.SemaphoreType.DMA,
            pltpu.SemaphoreType.DMA,
            pltpu.SemaphoreType.DMA,
            pltpu.SemaphoreType.DMA,
        ],
        compiler_params=pltpu.CompilerParams(needs_layout_passes=False,
                                             use_tc_tiling_on_sc=False,
                                             disable_bounds_checks=True),
        interpret=interpret,
    )


def kernel(patient_idx, gender, age_bin, query_symptoms, candidate_symptoms,
           symptom_emb, patient_emb, demog_W, demog_b, cf_bias,
           fusion_weights):
    w = jax.nn.softmax(fusion_weights.astype(jnp.float32))
    params = (jnp.zeros((_L,), jnp.float32)
              .at[0].set(w[0])
              .at[1].set(w[1] * 0.5)
              .at[2].set(cf_bias.astype(jnp.float32)[0]))
    pidx = patient_idx.astype(jnp.int32)
    qidx = query_symptoms.astype(jnp.int32)
    cidx = candidate_symptoms.astype(jnp.int32).reshape(_B, 2, _CH)
    return _make_call()(pidx, qidx, cidx,
                        symptom_emb.astype(jnp.float32),
                        patient_emb.astype(jnp.float32),
                        params)


# speedup vs baseline: 1.0187x; 1.0067x over previous
import jax
import jax.numpy as jnp
from jax import lax
from jax.experimental import pallas as pl
from jax.experimental.pallas import tpu as pltpu
from jax.experimental.pallas import tpu_sc as plsc

_B, _Q, _C, _D = 4096, 50, 200, 32
_NC, _NS, _L = 2, 16, 16
_NW = _NC * _NS
_RPW = _B // _NW
_CH0, _CH1 = 128, 72

_PASS1 = (0, 16, 32, 48, 64, 80, 96)
_PASS2 = (112, 128, 144, 160, 176, 184)


def _splat(i):
    return jnp.full((_L,), i, jnp.int32)


def _rsqrt(n):
    i = plsc.bitcast(n, jnp.int32)
    i = jnp.full((_L,), 0x5F3759DF, jnp.int32) - lax.shift_right_logical(i, 1)
    y = plsc.bitcast(i, jnp.float32)
    for _ in range(3):
        y = y * (1.5 - 0.5 * n * y * y)
    return y


_NBUF = 4


def _body(pidx_hbm, qidx_hbm, cidx_hbm, sym_hbm, pat_hbm, par_hbm, out_hbm,
          pidx_v, qidx_v, cidx_v, prow_v, qbuf_v, cbuf_v, outb_v,
          par_v, red_v, sem0, sem1, sem2, sem3):
    sems = (sem0, sem1, sem2, sem3)
    wid = lax.axis_index("s") * _NC + lax.axis_index("c")
    base = wid * _RPW

    pltpu.sync_copy(par_hbm, par_v)
    pltpu.sync_copy(pidx_hbm.at[pl.ds(base, _RPW)], pidx_v)
    pltpu.sync_copy(qidx_hbm.at[pl.ds(base, _RPW)], qidx_v)
    pltpu.sync_copy(cidx_hbm.at[pl.ds(base, _RPW)], cidx_v)
    pltpu.async_copy(pat_hbm.at[pidx_v], prow_v, sem0).wait()

    par = par_v[...]
    w0v = lax.broadcast(par[0], (_L,))
    w1hv = lax.broadcast(par[1], (_L,))
    biasv = lax.broadcast(par[2], (_L,))
    iota = lax.iota(jnp.int32, _L)
    zf = jnp.zeros((_L,), jnp.float32)

    def issue_row(b, slot):
        pltpu.async_copy(sym_hbm.at[qidx_v.at[b]], qbuf_v.at[slot],
                         sems[slot])
        pltpu.async_copy(sym_hbm.at[cidx_v.at[b, pl.ds(0, _CH0)]],
                         cbuf_v.at[slot, pl.ds(0, _CH0)], sems[slot])
        pltpu.async_copy(sym_hbm.at[cidx_v.at[b, pl.ds(_CH0, _CH1)]],
                         cbuf_v.at[slot, pl.ds(_CH0, _CH1)], sems[slot])

    def wait_row(b, slot):
        pltpu.make_async_copy(sym_hbm.at[qidx_v.at[b]],
                              qbuf_v.at[slot], sems[slot]).wait()
        pltpu.make_async_copy(sym_hbm.at[cidx_v.at[b, pl.ds(0, _CH0)]],
                              cbuf_v.at[slot, pl.ds(0, _CH0)],
                              sems[slot]).wait()
        pltpu.make_async_copy(sym_hbm.at[cidx_v.at[b, pl.ds(_CH0, _CH1)]],
                              cbuf_v.at[slot, pl.ds(_CH0, _CH1)],
                              sems[slot]).wait()

    def compute_row(b, slot):
        def qsum(j, acc):
            return (acc[0] + qbuf_v[slot, j, pl.ds(0, _L)],
                    acc[1] + qbuf_v[slot, j, pl.ds(_L, _L)])
        qlo, qhi = lax.fori_loop(0, _Q, qsum, (zf, zf))
        s = qlo * qlo + qhi * qhi
        red_v[0, pl.ds(_L, _L)] = zf
        for step in (8, 4, 2, 1):
            red_v[0, pl.ds(0, _L)] = s
            s = s + plsc.load_gather(red_v, [_splat(0), iota + step])
        nq = lax.broadcast(s[0], (_L,))
        rq = jnp.minimum(_rsqrt(nq), 1.0 / (_Q * 1e-12))
        red_v[0, pl.ds(0, _L)] = qlo * rq
        red_v[0, pl.ds(_L, _L)] = qhi * rq

        parv = _splat(slot)
        bsplat = lax.broadcast(b, (_L,))
        zsplat = _splat(0)
        for starts in (_PASS1, _PASS2):
            nch = len(starts)

            def tstep(t, carry):
                m = carry[-1]
                accs = carry[:-1]
                pv = plsc.load_gather(prow_v, [bsplat, m])
                qv = plsc.load_gather(red_v, [zsplat, m])
                out = []
                for k in range(nch):
                    ap, aq, an = accs[3 * k:3 * k + 3]
                    ev = plsc.load_gather(cbuf_v,
                                          [parv, iota + starts[k], m])
                    out += [ap + ev * pv, aq + ev * qv, an + ev * ev]
                return out + [lax.bitwise_and(m + 1, _D - 1)]

            accs = lax.fori_loop(0, _D, tstep, [zf] * (3 * nch) + [iota])
            for k in range(nch):
                ap, aq, an = accs[3 * k:3 * k + 3]
                cf = ap + biasv
                sig = 1.0 / (1.0 + jnp.exp(-cf))
                r = jnp.minimum(_rsqrt(an), 1.0e12)
                res = w0v * sig + w1hv * (aq * r) + w1hv
                outb_v[b, pl.ds(starts[k], _L)] = res

    for s in range(_NBUF):
        issue_row(s, s)

    def group_step(g, carry):
        for s in range(_NBUF):
            b = g * _NBUF + s
            wait_row(b, s)
            compute_row(b, s)

            @pl.when(g < _RPW // _NBUF - 1)
            def _(b=b, s=s):
                issue_row(b + _NBUF, s)
        return carry

    lax.fori_loop(0, _RPW // _NBUF, group_step, 0)
    pltpu.sync_copy(outb_v, out_hbm.at[pl.ds(base, _RPW)])


def _make_call(interpret=False):
    mesh = plsc.VectorSubcoreMesh(core_axis_name="c", subcore_axis_name="s")
    return pl.kernel(
        _body,
        out_type=jax.ShapeDtypeStruct((_B, _C), jnp.float32),
        mesh=mesh,
        scratch_types=[
            pltpu.VMEM((_RPW,), jnp.int32),
            pltpu.VMEM((_RPW, _Q), jnp.int32),
            pltpu.VMEM((_RPW, _C), jnp.int32),
            pltpu.VMEM((_RPW, _D), jnp.float32),
            pltpu.VMEM((_NBUF, _Q, _D), jnp.float32),
            pltpu.VMEM((_NBUF, _C, _D), jnp.float32),
            pltpu.VMEM((_RPW, _C), jnp.float32),
            pltpu.VMEM((_L,), jnp.float32),
            pltpu.VMEM((1, 2 * _L), jnp.float32),
            pltpu.SemaphoreType.DMA,
            pltpu.SemaphoreType.DMA,
            pltpu.SemaphoreType.DMA,
            pltpu.SemaphoreType.DMA,
        ],
        compiler_params=pltpu.CompilerParams(needs_layout_passes=False,
                                             use_tc_tiling_on_sc=False,
                                             disable_bounds_checks=True),
        interpret=interpret,
    )


def kernel(patient_idx, gender, age_bin, query_symptoms, candidate_symptoms,
           symptom_emb, patient_emb, demog_W, demog_b, cf_bias,
           fusion_weights):
    w = jax.nn.softmax(fusion_weights.astype(jnp.float32))
    params = (jnp.zeros((_L,), jnp.float32)
              .at[0].set(w[0])
              .at[1].set(w[1] * 0.5)
              .at[2].set(cf_bias.astype(jnp.float32)[0]))
    pidx = patient_idx.astype(jnp.int32)
    qidx = query_symptoms.astype(jnp.int32)
    cidx = candidate_symptoms.astype(jnp.int32)
    return _make_call()(pidx, qidx, cidx,
                        symptom_emb.astype(jnp.float32),
                        patient_emb.astype(jnp.float32),
                        params)
